# baseline (speedup 1.0000x reference)
import functools

import jax
import jax.numpy as jnp
from jax.experimental import pallas as pl
from jax.experimental.pallas import tpu as pltpu

_THRESHOLD = 0.5
_BS = 8
_NSTREAM = 2


def _match_targets(t_ref, an_ref, A, H, W, T_per_b):
    tx = t_ref[1:2, :]
    ty = t_ref[2:3, :]
    tw = t_ref[3:4, :] * float(W)
    th = t_ref[4:5, :] * float(H)
    area_t = tw * th

    best_iou = None
    best_a = jnp.zeros_like(tx, dtype=jnp.int32)
    for k in range(A):
        aw = an_ref[k:k + 1, 0:1]
        ah = an_ref[k:k + 1, 1:2]
        inter = jnp.minimum(aw, tw) * jnp.minimum(ah, th)
        iou = inter / (aw * ah + area_t - inter)
        if k == 0:
            best_iou = iou
        else:
            upd = iou > best_iou
            best_a = jnp.where(upd, k, best_a)
            best_iou = jnp.where(upd, iou, best_iou)

    t_i = (tx * float(W)).astype(jnp.int32)
    t_j = (ty * float(H)).astype(jnp.int32)
    t_b = jax.lax.broadcasted_iota(jnp.int32, tx.shape, 1) // T_per_b
    hit = (best_iou > _THRESHOLD).astype(jnp.float32)
    lin = ((t_b * A + best_a) * H + t_j) * W + t_i
    return hit, lin


def _body(t_ref, an_ref, xa_ref, xb_ref, out_ref, *, A, H, W, C, T_total,
          T_per_b, n_elems, n_steps_per_stream):
    i = pl.program_id(0)
    n = pl.num_programs(0)
    rows = _BS * H * W
    n_grp = rows // 128

    hit, lin = _match_targets(t_ref, an_ref, A, H, W, T_per_b)

    plane_u = jax.lax.div(lin, H * W)
    cell_u = jax.lax.rem(lin, H * W)
    t_j = jax.lax.div(cell_u, W)
    t_i = jax.lax.rem(cell_u, W)

    row_iota = jax.lax.broadcasted_iota(jnp.int32, (H, T_total), 0)
    col_iota = jax.lax.broadcasted_iota(jnp.int32, (W, T_total), 0)
    oj_base = (row_iota == t_j)
    oi = (col_iota == t_i).astype(jnp.float32)

    s_sum = jnp.float32(0.0)
    for stream, x_ref in enumerate((xa_ref, xb_ref)):
        base_plane = (stream * n_steps_per_stream + i) * _BS

        pred = x_ref[:, :, :, 4].reshape(_BS * H, W)

        oj_rows = []
        for s in range(_BS):
            sel = hit * (plane_u == base_plane + s).astype(jnp.float32)
            oj_rows.append(oj_base.astype(jnp.float32) * sel)
        oj = jnp.concatenate(oj_rows, axis=0)
        cnt = jax.lax.dot_general(oj, oi, (((1,), (1,)), ((), ())),
                                  preferred_element_type=jnp.float32)
        gt = cnt > 0.0

        log_p = jnp.maximum(jnp.log(pred), -100.0)
        log_1p = jnp.maximum(jnp.log(1.0 - pred), -100.0)
        s_sum = s_sum + jnp.sum(jnp.where(gt, -log_p, -log_1p))

    acc = jnp.where(i == 0, 0.0, out_ref[0, 0]) + s_sum
    out_ref[0, 0] = jnp.where(i == n - 1, acc / float(n_elems), acc)


def kernel(output, anchors, targets):
    B, A, H, W, C = output.shape
    T = targets.shape[1]
    n_elems = B * A * H * W
    n_planes = B * A
    x4d = output.reshape(n_planes, H, W, C)
    tt = targets.reshape(B * T, 5).T
    n_steps = n_planes // (_BS * _NSTREAM)

    out = pl.pallas_call(
        functools.partial(_body, A=A, H=H, W=W, C=C, T_total=B * T,
                          T_per_b=T, n_elems=n_elems,
                          n_steps_per_stream=n_steps),
        grid=(n_steps,),
        in_specs=[
            pl.BlockSpec((5, B * T), lambda i: (0, 0)),
            pl.BlockSpec((A, 2), lambda i: (0, 0)),
            pl.BlockSpec((_BS, H, W, C), lambda i: (i, 0, 0, 0)),
            pl.BlockSpec((_BS, H, W, C), lambda i, n=n_steps: (i + n, 0, 0, 0)),
        ],
        out_specs=pl.BlockSpec(memory_space=pltpu.SMEM),
        out_shape=jax.ShapeDtypeStruct((1, 1), jnp.float32),
    )(tt, anchors, x4d, x4d)
    return out[0, 0]


# speedup vs baseline: 1.3742x; 1.3742x over previous
import functools

import jax
import jax.numpy as jnp
from jax.experimental import pallas as pl
from jax.experimental.pallas import tpu as pltpu

_THRESHOLD = 0.5
_BS = 8
_NSTREAM = 2


def _match_targets(t_ref, an_ref, A, H, W, T_per_b):
    tx = t_ref[1:2, :]
    ty = t_ref[2:3, :]
    tw = t_ref[3:4, :] * float(W)
    th = t_ref[4:5, :] * float(H)
    area_t = tw * th

    best_iou = None
    best_a = jnp.zeros_like(tx, dtype=jnp.int32)
    for k in range(A):
        aw = an_ref[k:k + 1, 0:1]
        ah = an_ref[k:k + 1, 1:2]
        inter = jnp.minimum(aw, tw) * jnp.minimum(ah, th)
        iou = inter / (aw * ah + area_t - inter)
        if k == 0:
            best_iou = iou
        else:
            upd = iou > best_iou
            best_a = jnp.where(upd, k, best_a)
            best_iou = jnp.where(upd, iou, best_iou)

    t_i = (tx * float(W)).astype(jnp.int32)
    t_j = (ty * float(H)).astype(jnp.int32)
    t_b = jax.lax.broadcasted_iota(jnp.int32, tx.shape, 1) // T_per_b
    hit = (best_iou > _THRESHOLD).astype(jnp.float32)
    lin = ((t_b * A + best_a) * H + t_j) * W + t_i
    return hit, lin


def _body(t_ref, an_ref, xa_ref, xb_ref, out_ref, ps_ref, *, A, H, W, C,
          T_total, T_per_b, n_elems, n_steps_per_stream):
    i = pl.program_id(0)
    n = pl.num_programs(0)
    rows = _BS * H * W
    n_grp = rows // 128

    hit, lin = _match_targets(t_ref, an_ref, A, H, W, T_per_b)

    plane_u = jax.lax.div(lin, H * W)
    cell_u = jax.lax.rem(lin, H * W)
    t_j = jax.lax.div(cell_u, W)
    t_i = jax.lax.rem(cell_u, W)

    row_iota = jax.lax.broadcasted_iota(jnp.int32, (H, T_total), 0)
    col_iota = jax.lax.broadcasted_iota(jnp.int32, (W, T_total), 0)
    oj_base = (row_iota == t_j)
    oi = (col_iota == t_i).astype(jnp.float32)

    s_sum = jnp.float32(0.0)
    for stream, x_ref in enumerate((xa_ref, xb_ref)):
        base_plane = (stream * n_steps_per_stream + i) * _BS

        ps_ref[:, :] = x_ref[:, :, :, 4].reshape(_BS * H, W)
        pred = ps_ref[:, :]

        oj_rows = []
        for s in range(_BS):
            sel = hit * (plane_u == base_plane + s).astype(jnp.float32)
            oj_rows.append(oj_base.astype(jnp.float32) * sel)
        oj = jnp.concatenate(oj_rows, axis=0)
        cnt = jax.lax.dot_general(oj, oi, (((1,), (1,)), ((), ())),
                                  preferred_element_type=jnp.float32)
        gt = cnt > 0.0

        log_p = jnp.maximum(jnp.log(pred), -100.0)
        log_1p = jnp.maximum(jnp.log(1.0 - pred), -100.0)
        s_sum = s_sum + jnp.sum(jnp.where(gt, -log_p, -log_1p))

    acc = jnp.where(i == 0, 0.0, out_ref[0, 0]) + s_sum
    out_ref[0, 0] = jnp.where(i == n - 1, acc / float(n_elems), acc)


def kernel(output, anchors, targets):
    B, A, H, W, C = output.shape
    T = targets.shape[1]
    n_elems = B * A * H * W
    n_planes = B * A
    x4d = output.reshape(n_planes, H, W, C)
    tt = targets.reshape(B * T, 5).T
    n_steps = n_planes // (_BS * _NSTREAM)

    out = pl.pallas_call(
        functools.partial(_body, A=A, H=H, W=W, C=C, T_total=B * T,
                          T_per_b=T, n_elems=n_elems,
                          n_steps_per_stream=n_steps),
        grid=(n_steps,),
        in_specs=[
            pl.BlockSpec((5, B * T), lambda i: (0, 0)),
            pl.BlockSpec((A, 2), lambda i: (0, 0)),
            pl.BlockSpec((_BS, H, W, C), lambda i: (i, 0, 0, 0)),
            pl.BlockSpec((_BS, H, W, C), lambda i, n=n_steps: (i + n, 0, 0, 0)),
        ],
        out_specs=pl.BlockSpec(memory_space=pltpu.SMEM),
        out_shape=jax.ShapeDtypeStruct((1, 1), jnp.float32),
        scratch_shapes=[pltpu.VMEM((_BS * H, W), jnp.float32)],
    )(tt, anchors, x4d, x4d)
    return out[0, 0]
